# baseline (speedup 1.0000x reference)
import functools

import jax
import jax.numpy as jnp
from jax import lax
from jax.experimental import pallas as pl
from jax.experimental.pallas import tpu as pltpu
from jax.experimental.pallas import tpu_sc as plsc

VOCAB = 1000000
EMBED_DIM = 64
SEQ = 200
BATCH = 4096
NUM_WORKERS = 32

_MESH = dict(core_axis_name="c", subcore_axis_name="s")


_WV = 256
_N_FULL = VOCAB // _WV
_TAIL = VOCAB - _N_FULL * _WV
_TAIL_WORKER = 5
_UPW = (_N_FULL + NUM_WORKERS - 1) // NUM_WORKERS


@functools.partial(
    pl.kernel,
    mesh=plsc.VectorSubcoreMesh(**_MESH),
    compiler_params=pltpu.CompilerParams(needs_layout_passes=False),
    out_type=jax.ShapeDtypeStruct((VOCAB // 2, 2 * EMBED_DIM), jnp.float32),
    scratch_types=[
        pltpu.VMEM((EMBED_DIM, _WV), jnp.float32),
        pltpu.VMEM((EMBED_DIM, _WV), jnp.float32),
        pltpu.VMEM((_WV // 2, 2 * EMBED_DIM), jnp.float32),
        pltpu.VMEM((_WV // 2, 2 * EMBED_DIM), jnp.float32),
        pltpu.SemaphoreType.DMA,
        pltpu.SemaphoreType.DMA,
    ],
)
def _relayout(emb_t, tail_blk, table_rm, eb0, eb1, ov0, ov1, in_sem, out_sem):
    wid = lax.axis_index("s") * 2 + lax.axis_index("c")
    iota = lax.iota(jnp.int32, 16)

    def n_of(u):
        return jnp.minimum(wid + NUM_WORKERS * u, _N_FULL - 1)

    def start_in(u, eb):
        v0 = n_of(u) * _WV
        for g in range(EMBED_DIM // 8):
            pltpu.async_copy(emb_t.at[pl.ds(8 * g, 8), pl.ds(v0, _WV)],
                             eb.at[pl.ds(8 * g, 8), :], in_sem)

    def wait_in(eb):
        for g in range(EMBED_DIM // 8):
            pltpu.make_async_copy(emb_t.at[pl.ds(0, 8), pl.ds(0, _WV)],
                                  eb.at[pl.ds(8 * g, 8), :], in_sem).wait()

    def start_out(u, ov):
        pltpu.async_copy(ov, table_rm.at[pl.ds(n_of(u) * (_WV // 2), _WV // 2)],
                         out_sem)

    def wait_out(ov):
        pltpu.make_async_copy(
            ov, table_rm.at[pl.ds(0, _WV // 2)], out_sem).wait()

    def transpose(eb, ov):
        @plsc.parallel_loop(0, _WV // 16, unroll=2)
        def _body(mt):
            vv = iota + 16 * mt
            row_m = lax.shift_right_logical(vv, 1)
            colb_m = (vv & 1) * EMBED_DIM
            for dd in range(EMBED_DIM):
                vec = eb[dd, pl.ds(16 * mt, 16)]
                plsc.store_scatter(ov, [row_m, colb_m + dd], vec)

    def unit(u, g, eb, ov, eb_next):
        start_in(u + 1, eb_next)
        wait_in(eb)

        @pl.when(g >= 1)
        def _():
            wait_out(ov)

        transpose(eb, ov)
        start_out(u, ov)

    start_in(0, eb0)

    def pair_body(g, carry):
        unit(2 * g, g, eb0, ov0, eb1)
        unit(2 * g + 1, g, eb1, ov1, eb0)
        return carry

    lax.fori_loop(0, _UPW // 2, pair_body, 0)
    u_last = _UPW - 1
    wait_in(eb0)
    wait_out(ov0)
    transpose(eb0, ov0)
    start_out(u_last, ov0)
    wait_out(ov0)
    wait_out(ov1)

    @pl.when(wid == _TAIL_WORKER)
    def _tail():
        pltpu.sync_copy(tail_blk, ov0.at[pl.ds(0, _TAIL // 2)])
        pltpu.sync_copy(ov0.at[pl.ds(0, _TAIL // 2)],
                        table_rm.at[pl.ds(_N_FULL * (_WV // 2), _TAIL // 2)])


_BW = 128
_GD = EMBED_DIM // 8


@functools.partial(
    pl.kernel,
    mesh=plsc.VectorSubcoreMesh(**_MESH),
    compiler_params=pltpu.CompilerParams(use_tc_tiling_on_sc=False,
                                        needs_layout_passes=False),
    out_type=jax.ShapeDtypeStruct((SEQ, _GD, BATCH // _BW, 8, _BW),
                                  jnp.float32),
    scratch_types=[
        pltpu.VMEM((SEQ * _BW,), jnp.int32),
        pltpu.VMEM((_BW, EMBED_DIM), jnp.float32),
        pltpu.VMEM((_BW, EMBED_DIM), jnp.float32),
        pltpu.VMEM((_GD, 8, _BW), jnp.float32),
        pltpu.VMEM((_GD, 8, _BW), jnp.float32),
        pltpu.SemaphoreType.DMA,
        pltpu.SemaphoreType.DMA,
    ],
)
def _gather(idx_wmaj, table, out, idxw_v, r0, r1, ob0, ob1, in_sem, out_sem):
    wid = lax.axis_index("s") * 2 + lax.axis_index("c")
    b0 = wid * _BW
    iota = lax.iota(jnp.int32, 16)
    rows = [iota + 16 * m for m in range(8)]

    pltpu.sync_copy(idx_wmaj.at[pl.ds(wid * (SEQ * _BW), SEQ * _BW)], idxw_v)

    def idx_ref(u):
        t = jnp.minimum(u, SEQ - 1)
        return idxw_v.at[pl.ds(t * _BW, _BW)]

    def start_gather(u, rv):
        pltpu.async_copy(table.at[idx_ref(u)], rv, in_sem)

    def wait_gather(rv):
        pltpu.make_async_copy(table.at[idx_ref(0)], rv, in_sem).wait()

    def start_out(u, ob):
        pltpu.async_copy(ob, out.at[u, :, wid], out_sem)

    def wait_out(ob):
        pltpu.make_async_copy(ob, out.at[0, :, wid], out_sem).wait()

    def transpose(rv, ob):
        @plsc.parallel_loop(0, EMBED_DIM, unroll=16)
        def _body(dd):
            g = lax.shift_right_logical(dd, 3)
            dj = dd & 7
            dvec = jnp.broadcast_to(dd, (16,))
            for m in range(8):
                vec = plsc.load_gather(rv, [rows[m], dvec])
                ob[g, dj, pl.ds(16 * m, 16)] = vec

    def unit(u, g, rv, ob, rv_next):
        start_gather(u + 1, rv_next)
        wait_gather(rv)

        @pl.when(g >= 1)
        def _():
            wait_out(ob)

        transpose(rv, ob)
        start_out(u, ob)

    start_gather(0, r0)

    def pair_body(g, carry):
        unit(2 * g, g, r0, ob0, r1)
        unit(2 * g + 1, g, r1, ob1, r0)
        return carry

    lax.fori_loop(0, SEQ // 2, pair_body, 0)
    wait_gather(r0)
    wait_out(ob0)
    wait_out(ob1)


def kernel(inputs, embeddings):
    emb_t = embeddings.T
    tail_blk = jnp.reshape(
        lax.slice(embeddings, (VOCAB - _TAIL, 0), (VOCAB, EMBED_DIM)),
        (_TAIL // 2, 2 * EMBED_DIM))
    table_rm = _relayout(emb_t, tail_blk)
    table = jnp.reshape(table_rm, (VOCAB, EMBED_DIM))
    idx_wmaj = jnp.reshape(
        jnp.transpose(jnp.reshape(inputs, (NUM_WORKERS, _BW, SEQ)), (0, 2, 1)),
        (BATCH * SEQ,))
    out5 = _gather(idx_wmaj, table)
    out = jnp.reshape(jnp.transpose(out5, (2, 4, 0, 1, 3)),
                      (BATCH, SEQ, EMBED_DIM))
    return out
